# baseline (speedup 1.0000x reference)
import functools

import jax
import jax.numpy as jnp
from jax import lax
from jax.experimental import pallas as pl
from jax.experimental.pallas import tpu as pltpu

_HD = 128
_NEG = -3.0e38


def _exp_acc(x):
    x = jnp.clip(x, -87.0, 88.0)
    k = jnp.floor(x * 1.4426950408889634 + 0.5)
    r = (x - k * 0.693359375) - k * (-2.12194440e-4)
    r2 = r * r
    p = (1.0 + r * (1.0 + r * (0.5 + r * (1.0 / 6.0 + r * (1.0 / 24.0
         + r * (1.0 / 120.0 + r * (1.0 / 720.0 + r * (1.0 / 5040.0))))))))
    ki = k.astype(jnp.int32)
    scale = lax.bitcast_convert_type(
        lax.shift_left(ki + 127, 23), jnp.float32)
    return p * scale


def _log1p_acc(y):
    s = y / (2.0 + y)
    t = s * s
    q = (1.0 + t * (1.0 / 3.0 + t * (1.0 / 5.0 + t * (1.0 / 7.0
         + t * (1.0 / 9.0 + t * (1.0 / 11.0 + t * (1.0 / 13.0)))))))
    return 2.0 * s * q


def _softplus_acc(x):
    return jnp.maximum(x, 0.0) + _log1p_acc(_exp_acc(-jnp.abs(x)))


def _ln(x, g, b, eps=1e-5):
    mu = jnp.mean(x, axis=-1, keepdims=True)
    var = jnp.mean((x - mu) ** 2, axis=-1, keepdims=True)
    return (x - mu) / jnp.sqrt(var + eps) * g + b


def _dot(a, b, precision=lax.Precision.HIGHEST):
    return jax.lax.dot_general(a, b, (((1,), (0,)), ((), ())),
                               preferred_element_type=jnp.float32,
                               precision=precision)


def _k1_body(x_ref, g_ref, b_ref, w_ref, bias_ref, o_ref):
    y = _ln(x_ref[...], g_ref[...], b_ref[...])
    o_ref[...] = _dot(y, w_ref[...]) + bias_ref[...]


def _ln_qkv(x, gamma, beta, Wqkv, bqkv, bm, bn):
    T, M = x.shape
    P = Wqkv.shape[1]
    grid = (P // bn, T // bm)
    return pl.pallas_call(
        _k1_body,
        grid=grid,
        in_specs=[
            pl.BlockSpec((bm, M), lambda j, i: (i, 0)),
            pl.BlockSpec((1, M), lambda j, i: (0, 0)),
            pl.BlockSpec((1, M), lambda j, i: (0, 0)),
            pl.BlockSpec((M, bn), lambda j, i: (0, j)),
            pl.BlockSpec((1, bn), lambda j, i: (0, j)),
        ],
        out_specs=pl.BlockSpec((bm, bn), lambda j, i: (i, j)),
        out_shape=jax.ShapeDtypeStruct((T, P), jnp.float32),
        compiler_params=pltpu.CompilerParams(
            dimension_semantics=("parallel", "parallel")),
    )(x, gamma, beta, Wqkv, bqkv)


def _attn_body(q_ref, k_ref, v_ref, o_ref, *, scale):
    q = q_ref[0].astype(jnp.bfloat16)
    kt = k_ref[0].astype(jnp.bfloat16)
    v = v_ref[0].astype(jnp.bfloat16)
    s = _dot(q, kt, precision=lax.Precision.DEFAULT) * scale
    m = jnp.max(s, axis=-1, keepdims=True)
    u = jnp.exp(s - m)
    S = jnp.sum(u, axis=-1, keepdims=True)
    o_ref[0] = _dot(u.astype(jnp.bfloat16), v,
                    precision=lax.Precision.DEFAULT) / S


def _attention(qh, kth, vh, bq):
    NH, N, HD = qh.shape
    NG = kth.shape[0]
    grid = (NH, N // bq)
    return pl.pallas_call(
        functools.partial(_attn_body, scale=HD ** (-0.5)),
        grid=grid,
        in_specs=[
            pl.BlockSpec((1, bq, HD), lambda h, i: (h, i, 0)),
            pl.BlockSpec((1, HD, N), lambda h, i: (h % NG, 0, 0)),
            pl.BlockSpec((1, N, HD), lambda h, i: (h % NG, 0, 0)),
        ],
        out_specs=pl.BlockSpec((1, bq, HD), lambda h, i: (h, i, 0)),
        out_shape=jax.ShapeDtypeStruct((NH, N, HD), jnp.float32),
        compiler_params=pltpu.CompilerParams(
            dimension_semantics=("parallel", "parallel")),
    )(qh, kth, vh)


def _k3_body(a_ref, w_ref, b_ref, x_ref, o_ref):
    o_ref[...] = _dot(a_ref[...], w_ref[...]) + b_ref[...] + x_ref[...]


def _proj_residual(ao, Wo, bo, x, bm, bn):
    T, M = ao.shape
    grid = (M // bn, T // bm)
    return pl.pallas_call(
        _k3_body,
        grid=grid,
        in_specs=[
            pl.BlockSpec((bm, M), lambda j, i: (i, 0)),
            pl.BlockSpec((M, bn), lambda j, i: (0, j)),
            pl.BlockSpec((1, bn), lambda j, i: (0, j)),
            pl.BlockSpec((bm, bn), lambda j, i: (i, j)),
        ],
        out_specs=pl.BlockSpec((bm, bn), lambda j, i: (i, j)),
        out_shape=jax.ShapeDtypeStruct((T, M), jnp.float32),
        compiler_params=pltpu.CompilerParams(
            dimension_semantics=("parallel", "parallel")),
    )(ao, Wo, bo, x)


def _cv_sq(v):
    mean = jnp.mean(v)
    std = jnp.sqrt(jnp.mean((v - mean) ** 2))
    return std / (mean + 1e-6)


def _k4_body(x1_ref, g_ref, b_ref, wg_ref, bg_ref, wn_ref, bn_ref, noise_ref,
             z_ref, gates_ref, idx_ref, sm_ref, lmoe_ref, hv_ref, ns_ref,
             imp_ref, load_ref):
    step = pl.program_id(0)
    nsteps = pl.num_programs(0)
    z = _ln(x1_ref[...], g_ref[...], b_ref[...])
    z_ref[...] = z
    E = noise_ref.shape[1]
    zb = z.astype(jnp.bfloat16)
    logits = (_dot(zb, wg_ref[...].astype(jnp.bfloat16),
                   precision=lax.Precision.DEFAULT) + bg_ref[...])[:, :E]
    ns = jax.nn.softplus(
        (_dot(zb, wn_ref[...].astype(jnp.bfloat16),
              precision=lax.Precision.DEFAULT) + bn_ref[...])[:, :E])
    Hv = logits + noise_ref[...] * ns
    hv_ref[...] = Hv
    ns_ref[...] = ns
    T, E = Hv.shape
    iota = lax.broadcasted_iota(jnp.int32, (T, E), 1)
    m1 = jnp.max(Hv, axis=-1, keepdims=True)
    i1 = jnp.min(jnp.where(Hv == m1, iota, E), axis=-1, keepdims=True)
    v2 = jnp.where(iota == i1, _NEG, Hv)
    m2 = jnp.max(v2, axis=-1, keepdims=True)
    i2 = jnp.min(jnp.where(v2 == m2, iota, E), axis=-1, keepdims=True)
    v3 = jnp.where(iota == i2, _NEG, v2)
    m3 = jnp.max(v3, axis=-1, keepdims=True)
    e2 = jnp.exp(m2 - m1)
    g1 = 1.0 / (1.0 + e2)
    g2 = e2 * g1
    gates = (jnp.where(iota == i1, g1, 0.0)
             + jnp.where(iota == i2, g2, 0.0))
    gates_ref[...] = gates
    idx_ref[...] = jnp.concatenate(
        [i1, i2, jnp.zeros((T, E - 2), jnp.int32)], axis=-1)
    sm_ref[...] = jnp.concatenate(
        [g1, g2, jnp.zeros((T, E - 2), jnp.float32)], axis=-1)
    importance = jnp.sum(gates, axis=0, keepdims=True)
    psi = jnp.where(Hv > m2, m2, jnp.where(Hv <= m3, m3, Hv))
    P = 0.5 * (1.0 + lax.erf((logits - psi) / (ns * jnp.sqrt(2.0))))
    load = jnp.sum(P, axis=0, keepdims=True)

    @pl.when(step == 0)
    def _():
        imp_ref[...] = jnp.zeros_like(imp_ref)
        load_ref[...] = jnp.zeros_like(load_ref)

    imp_ref[...] += importance
    load_ref[...] += load

    @pl.when(step == nsteps - 1)
    def _():
        lmoe = 0.01 * _cv_sq(imp_ref[...]) + 0.01 * _cv_sq(load_ref[...])
        lmoe_ref[...] = lmoe[None, None]


def _gating(x1, gamma, beta, Wg, bg, Wn, bn, noise, bt):
    T, M = x1.shape
    EP = Wg.shape[1]
    E = noise.shape[1]
    grid = (T // bt,)
    return pl.pallas_call(
        _k4_body,
        grid=grid,
        in_specs=[
            pl.BlockSpec((bt, M), lambda i: (i, 0)),
            pl.BlockSpec((1, M), lambda i: (0, 0)),
            pl.BlockSpec((1, M), lambda i: (0, 0)),
            pl.BlockSpec((M, EP), lambda i: (0, 0)),
            pl.BlockSpec((1, EP), lambda i: (0, 0)),
            pl.BlockSpec((M, EP), lambda i: (0, 0)),
            pl.BlockSpec((1, EP), lambda i: (0, 0)),
            pl.BlockSpec((bt, E), lambda i: (i, 0)),
        ],
        out_specs=[
            pl.BlockSpec((bt, M), lambda i: (i, 0)),
            pl.BlockSpec((bt, E), lambda i: (i, 0)),
            pl.BlockSpec((bt, E), lambda i: (i, 0)),
            pl.BlockSpec((bt, E), lambda i: (i, 0)),
            pl.BlockSpec((1, 1), lambda i: (0, 0)),
            pl.BlockSpec((bt, E), lambda i: (i, 0)),
            pl.BlockSpec((bt, E), lambda i: (i, 0)),
        ],
        out_shape=[
            jax.ShapeDtypeStruct((T, M), jnp.float32),
            jax.ShapeDtypeStruct((T, E), jnp.float32),
            jax.ShapeDtypeStruct((T, E), jnp.int32),
            jax.ShapeDtypeStruct((T, E), jnp.float32),
            jax.ShapeDtypeStruct((1, 1), jnp.float32),
            jax.ShapeDtypeStruct((T, E), jnp.float32),
            jax.ShapeDtypeStruct((T, E), jnp.float32),
        ],
        scratch_shapes=[
            pltpu.VMEM((1, E), jnp.float32),
            pltpu.VMEM((1, E), jnp.float32),
        ],
        compiler_params=pltpu.CompilerParams(
            dimension_semantics=("arbitrary",)),
    )(x1, gamma, beta, Wg, bg, Wn, bn, noise)


def _xv(z, V, bV, bm, bn):
    T, M = z.shape
    DH = V.shape[1]
    grid = (DH // bn, T // bm)
    return pl.pallas_call(
        lambda z_ref, v_ref, b_ref, o_ref: o_ref.__setitem__(
            ..., _dot(z_ref[...], v_ref[...]) + b_ref[...]),
        grid=grid,
        in_specs=[
            pl.BlockSpec((bm, M), lambda j, i: (i, 0)),
            pl.BlockSpec((M, bn), lambda j, i: (0, j)),
            pl.BlockSpec((1, bn), lambda j, i: (0, j)),
        ],
        out_specs=pl.BlockSpec((bm, bn), lambda j, i: (i, j)),
        out_shape=jax.ShapeDtypeStruct((T, DH), jnp.float32),
        compiler_params=pltpu.CompilerParams(
            dimension_semantics=("parallel", "parallel")),
    )(z, V, bV)


def _k6_body(z_ref, we_ref, bwe_ref, xv_ref, w2_ref, bw2_ref, gates_ref,
             x1_ref, o_ref, *, nj):
    e = pl.program_id(1)
    j = pl.program_id(2)

    @pl.when(jnp.logical_and(e == 0, j == 0))
    def _():
        o_ref[...] = x1_ref[...]

    T, E = gates_ref.shape
    iota = lax.broadcasted_iota(jnp.int32, (T, E), 1)
    gcol = jnp.sum(jnp.where(iota == e, gates_ref[...], 0.0),
                   axis=-1, keepdims=True)
    h = _dot(z_ref[...], we_ref[0]) + bwe_ref[0]
    a = h * jax.nn.sigmoid(h) * xv_ref[...]
    part = _dot(a, w2_ref[...])

    @pl.when(j == 0)
    def _():
        o_ref[...] += bw2_ref[...] * gcol

    o_ref[...] += part * gcol


def _experts_dense(z, We, bWe, xv, W2, bW2, gates, x1, bt, bj):
    T, M = z.shape
    E, _, DH = We.shape
    grid = (T // bt, E, DH // bj)
    return pl.pallas_call(
        functools.partial(_k6_body, nj=DH // bj),
        grid=grid,
        in_specs=[
            pl.BlockSpec((bt, M), lambda t, e, j: (t, 0)),
            pl.BlockSpec((1, M, bj), lambda t, e, j: (e, 0, j)),
            pl.BlockSpec((1, 1, bj), lambda t, e, j: (e, 0, j)),
            pl.BlockSpec((bt, bj), lambda t, e, j: (t, j)),
            pl.BlockSpec((bj, M), lambda t, e, j: (j, 0)),
            pl.BlockSpec((1, M), lambda t, e, j: (0, 0)),
            pl.BlockSpec((bt, E), lambda t, e, j: (t, 0)),
            pl.BlockSpec((bt, M), lambda t, e, j: (t, 0)),
        ],
        out_specs=pl.BlockSpec((bt, M), lambda t, e, j: (t, 0)),
        out_shape=jax.ShapeDtypeStruct((T, M), jnp.float32),
        compiler_params=pltpu.CompilerParams(
            dimension_semantics=("arbitrary", "arbitrary", "arbitrary")),
    )(z, We, bWe, xv, W2, bW2, gates, x1)


def kernel(x, gamma1, beta1, Wq, bq, Wk, bk, Wv, bv, Wo, bo, gamma2, beta2,
           Wg, bg, Wn, bn, We, bWe, V, bV, W2, bW2):
    B, N, M = x.shape
    T = B * N
    NH = M // _HD
    NG = Wk.shape[1] // _HD
    E = Wg.shape[1]
    DH = We.shape[2]

    xf = x.reshape(T, M)
    g1r = gamma1.reshape(1, M)
    b1r = beta1.reshape(1, M)

    Wqkv = jnp.concatenate([Wq, Wk, Wv], axis=1)
    bqkv = jnp.concatenate([bq, bk, bv]).reshape(1, -1)

    qkv = _ln_qkv(xf, g1r, b1r, Wqkv, bqkv, bm=min(256, T),
                  bn=min(1024, Wqkv.shape[1]))
    q = qkv[:, :M].reshape(N, NH, _HD).transpose(1, 0, 2)
    k = qkv[:, M:M + NG * _HD].reshape(N, NG, _HD).transpose(1, 2, 0)
    v = qkv[:, M + NG * _HD:].reshape(N, NG, _HD).transpose(1, 0, 2)

    ao = _attention(q, k, v, bq=min(256, N))
    ao = ao.transpose(1, 0, 2).reshape(T, M)

    x1 = _proj_residual(ao, Wo, bo.reshape(1, M), xf, bm=min(256, T),
                        bn=min(1024, M))

    noise = jax.random.normal(jax.random.key(42), (T, E), dtype=jnp.float32)
    EP = 128
    pad = jnp.zeros((M, EP - E), jnp.float32)
    padb = jnp.zeros((EP - E,), jnp.float32)
    z, gates, idx8, sm8, lmoe, _hv, _ns = _gating(
        x1, gamma2.reshape(1, M), beta2.reshape(1, M),
        jnp.concatenate([Wg, pad], 1),
        jnp.concatenate([bg, padb]).reshape(1, EP),
        jnp.concatenate([Wn, pad], 1),
        jnp.concatenate([bn, padb]).reshape(1, EP), noise, bt=min(512, T))

    xv = _xv(z, V, bV.reshape(1, DH), bm=min(256, T), bn=min(1024, DH))

    out = _experts_dense(z, We, bWe.reshape(E, 1, DH), xv, W2,
                         bW2.reshape(1, M), gates, x1,
                         bt=min(512, T), bj=min(512, DH))

    return out.reshape(B, N, M), lmoe[0, 0]


# speedup vs baseline: 2.8248x; 2.8248x over previous
import functools

import jax
import jax.numpy as jnp
from jax import lax
from jax.experimental import pallas as pl
from jax.experimental.pallas import tpu as pltpu

_HD = 128
_NEG = -3.0e38


def _exp_acc(x):
    x = jnp.clip(x, -87.0, 88.0)
    k = jnp.floor(x * 1.4426950408889634 + 0.5)
    r = (x - k * 0.693359375) - k * (-2.12194440e-4)
    r2 = r * r
    p = (1.0 + r * (1.0 + r * (0.5 + r * (1.0 / 6.0 + r * (1.0 / 24.0
         + r * (1.0 / 120.0 + r * (1.0 / 720.0 + r * (1.0 / 5040.0))))))))
    ki = k.astype(jnp.int32)
    scale = lax.bitcast_convert_type(
        lax.shift_left(ki + 127, 23), jnp.float32)
    return p * scale


def _log1p_acc(y):
    s = y / (2.0 + y)
    t = s * s
    q = (1.0 + t * (1.0 / 3.0 + t * (1.0 / 5.0 + t * (1.0 / 7.0
         + t * (1.0 / 9.0 + t * (1.0 / 11.0 + t * (1.0 / 13.0)))))))
    return 2.0 * s * q


def _softplus_acc(x):
    return jnp.maximum(x, 0.0) + _log1p_acc(_exp_acc(-jnp.abs(x)))


def _ln(x, g, b, eps=1e-5):
    mu = jnp.mean(x, axis=-1, keepdims=True)
    var = jnp.mean((x - mu) ** 2, axis=-1, keepdims=True)
    return (x - mu) / jnp.sqrt(var + eps) * g + b


def _dot(a, b, precision=lax.Precision.HIGHEST):
    return jax.lax.dot_general(a, b, (((1,), (0,)), ((), ())),
                               preferred_element_type=jnp.float32,
                               precision=precision)


def _k1_body(x_ref, g_ref, b_ref, w_ref, bias_ref, o_ref):
    y = _ln(x_ref[...], g_ref[...], b_ref[...])
    o_ref[...] = _dot(y, w_ref[...]) + bias_ref[...]


def _ln_qkv(x, gamma, beta, Wqkv, bqkv, bm, bn):
    T, M = x.shape
    P = Wqkv.shape[1]
    grid = (P // bn, T // bm)
    return pl.pallas_call(
        _k1_body,
        grid=grid,
        in_specs=[
            pl.BlockSpec((bm, M), lambda j, i: (i, 0)),
            pl.BlockSpec((1, M), lambda j, i: (0, 0)),
            pl.BlockSpec((1, M), lambda j, i: (0, 0)),
            pl.BlockSpec((M, bn), lambda j, i: (0, j)),
            pl.BlockSpec((1, bn), lambda j, i: (0, j)),
        ],
        out_specs=pl.BlockSpec((bm, bn), lambda j, i: (i, j)),
        out_shape=jax.ShapeDtypeStruct((T, P), jnp.float32),
        compiler_params=pltpu.CompilerParams(
            dimension_semantics=("parallel", "parallel")),
    )(x, gamma, beta, Wqkv, bqkv)


def _attn_body(q_ref, k_ref, v_ref, o_ref, *, scale):
    q = q_ref[0].astype(jnp.bfloat16)
    kt = k_ref[0].astype(jnp.bfloat16)
    v = v_ref[0].astype(jnp.bfloat16)
    s = _dot(q, kt, precision=lax.Precision.DEFAULT) * scale
    m = jnp.max(s, axis=-1, keepdims=True)
    u = jnp.exp(s - m)
    S = jnp.sum(u, axis=-1, keepdims=True)
    o_ref[0] = _dot(u.astype(jnp.bfloat16), v,
                    precision=lax.Precision.DEFAULT) / S


def _attention(qh, kth, vh, bq):
    NH, N, HD = qh.shape
    NG = kth.shape[0]
    grid = (NH, N // bq)
    return pl.pallas_call(
        functools.partial(_attn_body, scale=HD ** (-0.5)),
        grid=grid,
        in_specs=[
            pl.BlockSpec((1, bq, HD), lambda h, i: (h, i, 0)),
            pl.BlockSpec((1, HD, N), lambda h, i: (h % NG, 0, 0)),
            pl.BlockSpec((1, N, HD), lambda h, i: (h % NG, 0, 0)),
        ],
        out_specs=pl.BlockSpec((1, bq, HD), lambda h, i: (h, i, 0)),
        out_shape=jax.ShapeDtypeStruct((NH, N, HD), jnp.float32),
        compiler_params=pltpu.CompilerParams(
            dimension_semantics=("parallel", "parallel")),
    )(qh, kth, vh)


def _k3_body(a_ref, w_ref, b_ref, x_ref, o_ref):
    o_ref[...] = _dot(a_ref[...], w_ref[...]) + b_ref[...] + x_ref[...]


def _proj_residual(ao, Wo, bo, x, bm, bn):
    T, M = ao.shape
    grid = (M // bn, T // bm)
    return pl.pallas_call(
        _k3_body,
        grid=grid,
        in_specs=[
            pl.BlockSpec((bm, M), lambda j, i: (i, 0)),
            pl.BlockSpec((M, bn), lambda j, i: (0, j)),
            pl.BlockSpec((1, bn), lambda j, i: (0, j)),
            pl.BlockSpec((bm, bn), lambda j, i: (i, j)),
        ],
        out_specs=pl.BlockSpec((bm, bn), lambda j, i: (i, j)),
        out_shape=jax.ShapeDtypeStruct((T, M), jnp.float32),
        compiler_params=pltpu.CompilerParams(
            dimension_semantics=("parallel", "parallel")),
    )(ao, Wo, bo, x)


def _cv_sq(v):
    mean = jnp.mean(v)
    std = jnp.sqrt(jnp.mean((v - mean) ** 2))
    return std / (mean + 1e-6)


def _k4_body(x1_ref, g_ref, b_ref, wg_ref, bg_ref, wn_ref, bn_ref, noise_ref,
             z_ref, gates_ref, idx_ref, sm_ref, lmoe_ref, hv_ref, ns_ref,
             imp_ref, load_ref):
    step = pl.program_id(0)
    nsteps = pl.num_programs(0)
    z = _ln(x1_ref[...], g_ref[...], b_ref[...])
    z_ref[...] = z
    E = noise_ref.shape[1]
    zb = z.astype(jnp.bfloat16)
    logits = (_dot(zb, wg_ref[...].astype(jnp.bfloat16),
                   precision=lax.Precision.DEFAULT) + bg_ref[...])[:, :E]
    ns = jax.nn.softplus(
        (_dot(zb, wn_ref[...].astype(jnp.bfloat16),
              precision=lax.Precision.DEFAULT) + bn_ref[...])[:, :E])
    Hv = logits + noise_ref[...] * ns
    hv_ref[...] = Hv
    ns_ref[...] = ns
    T, E = Hv.shape
    iota = lax.broadcasted_iota(jnp.int32, (T, E), 1)
    m1 = jnp.max(Hv, axis=-1, keepdims=True)
    i1 = jnp.min(jnp.where(Hv == m1, iota, E), axis=-1, keepdims=True)
    v2 = jnp.where(iota == i1, _NEG, Hv)
    m2 = jnp.max(v2, axis=-1, keepdims=True)
    i2 = jnp.min(jnp.where(v2 == m2, iota, E), axis=-1, keepdims=True)
    v3 = jnp.where(iota == i2, _NEG, v2)
    m3 = jnp.max(v3, axis=-1, keepdims=True)
    e2 = jnp.exp(m2 - m1)
    g1 = 1.0 / (1.0 + e2)
    g2 = e2 * g1
    gates = (jnp.where(iota == i1, g1, 0.0)
             + jnp.where(iota == i2, g2, 0.0))
    gates_ref[...] = gates
    idx_ref[...] = jnp.concatenate(
        [i1, i2, jnp.zeros((T, E - 2), jnp.int32)], axis=-1)
    sm_ref[...] = jnp.concatenate(
        [g1, g2, jnp.zeros((T, E - 2), jnp.float32)], axis=-1)
    importance = jnp.sum(gates, axis=0, keepdims=True)
    psi = jnp.where(Hv > m2, m2, jnp.where(Hv <= m3, m3, Hv))
    P = 0.5 * (1.0 + lax.erf((logits - psi) / (ns * jnp.sqrt(2.0))))
    load = jnp.sum(P, axis=0, keepdims=True)

    @pl.when(step == 0)
    def _():
        imp_ref[...] = jnp.zeros_like(imp_ref)
        load_ref[...] = jnp.zeros_like(load_ref)

    imp_ref[...] += importance
    load_ref[...] += load

    @pl.when(step == nsteps - 1)
    def _():
        lmoe = 0.01 * _cv_sq(imp_ref[...]) + 0.01 * _cv_sq(load_ref[...])
        lmoe_ref[...] = lmoe[None, None]


def _gating(x1, gamma, beta, Wg, bg, Wn, bn, noise, bt):
    T, M = x1.shape
    EP = Wg.shape[1]
    E = noise.shape[1]
    grid = (T // bt,)
    return pl.pallas_call(
        _k4_body,
        grid=grid,
        in_specs=[
            pl.BlockSpec((bt, M), lambda i: (i, 0)),
            pl.BlockSpec((1, M), lambda i: (0, 0)),
            pl.BlockSpec((1, M), lambda i: (0, 0)),
            pl.BlockSpec((M, EP), lambda i: (0, 0)),
            pl.BlockSpec((1, EP), lambda i: (0, 0)),
            pl.BlockSpec((M, EP), lambda i: (0, 0)),
            pl.BlockSpec((1, EP), lambda i: (0, 0)),
            pl.BlockSpec((bt, E), lambda i: (i, 0)),
        ],
        out_specs=[
            pl.BlockSpec((bt, M), lambda i: (i, 0)),
            pl.BlockSpec((bt, E), lambda i: (i, 0)),
            pl.BlockSpec((bt, E), lambda i: (i, 0)),
            pl.BlockSpec((bt, E), lambda i: (i, 0)),
            pl.BlockSpec((1, 1), lambda i: (0, 0)),
            pl.BlockSpec((bt, E), lambda i: (i, 0)),
            pl.BlockSpec((bt, E), lambda i: (i, 0)),
        ],
        out_shape=[
            jax.ShapeDtypeStruct((T, M), jnp.float32),
            jax.ShapeDtypeStruct((T, E), jnp.float32),
            jax.ShapeDtypeStruct((T, E), jnp.int32),
            jax.ShapeDtypeStruct((T, E), jnp.float32),
            jax.ShapeDtypeStruct((1, 1), jnp.float32),
            jax.ShapeDtypeStruct((T, E), jnp.float32),
            jax.ShapeDtypeStruct((T, E), jnp.float32),
        ],
        scratch_shapes=[
            pltpu.VMEM((1, E), jnp.float32),
            pltpu.VMEM((1, E), jnp.float32),
        ],
        compiler_params=pltpu.CompilerParams(
            dimension_semantics=("arbitrary",)),
    )(x1, gamma, beta, Wg, bg, Wn, bn, noise)


def _xv(z, V, bV, bm, bn):
    T, M = z.shape
    DH = V.shape[1]
    grid = (DH // bn, T // bm)
    return pl.pallas_call(
        lambda z_ref, v_ref, b_ref, o_ref: o_ref.__setitem__(
            ..., _dot(z_ref[...].astype(jnp.bfloat16),
                      v_ref[...].astype(jnp.bfloat16),
                      precision=lax.Precision.DEFAULT) + b_ref[...]),
        grid=grid,
        in_specs=[
            pl.BlockSpec((bm, M), lambda j, i: (i, 0)),
            pl.BlockSpec((M, bn), lambda j, i: (0, j)),
            pl.BlockSpec((1, bn), lambda j, i: (0, j)),
        ],
        out_specs=pl.BlockSpec((bm, bn), lambda j, i: (i, j)),
        out_shape=jax.ShapeDtypeStruct((T, DH), jnp.float32),
        compiler_params=pltpu.CompilerParams(
            dimension_semantics=("parallel", "parallel")),
    )(z, V, bV)


def _k6_body(z_ref, we_ref, bwe_ref, xv_ref, w2_ref, bw2_ref, gates_ref,
             x1_ref, o_ref, *, nj):
    e = pl.program_id(1)
    j = pl.program_id(2)

    @pl.when(jnp.logical_and(e == 0, j == 0))
    def _():
        o_ref[...] = x1_ref[...]

    T, E = gates_ref.shape
    iota = lax.broadcasted_iota(jnp.int32, (T, E), 1)
    gcol = jnp.sum(jnp.where(iota == e, gates_ref[...], 0.0),
                   axis=-1, keepdims=True)
    h = _dot(z_ref[...].astype(jnp.bfloat16), we_ref[0].astype(jnp.bfloat16),
             precision=lax.Precision.DEFAULT) + bwe_ref[0]
    a = h * jax.nn.sigmoid(h) * xv_ref[...]
    part = _dot(a.astype(jnp.bfloat16), w2_ref[...].astype(jnp.bfloat16),
                precision=lax.Precision.DEFAULT)

    @pl.when(j == 0)
    def _():
        o_ref[...] += bw2_ref[...] * gcol

    o_ref[...] += part * gcol


def _experts_dense(z, We, bWe, xv, W2, bW2, gates, x1, bt, bj):
    T, M = z.shape
    E, _, DH = We.shape
    grid = (T // bt, E, DH // bj)
    return pl.pallas_call(
        functools.partial(_k6_body, nj=DH // bj),
        grid=grid,
        in_specs=[
            pl.BlockSpec((bt, M), lambda t, e, j: (t, 0)),
            pl.BlockSpec((1, M, bj), lambda t, e, j: (e, 0, j)),
            pl.BlockSpec((1, 1, bj), lambda t, e, j: (e, 0, j)),
            pl.BlockSpec((bt, bj), lambda t, e, j: (t, j)),
            pl.BlockSpec((bj, M), lambda t, e, j: (j, 0)),
            pl.BlockSpec((1, M), lambda t, e, j: (0, 0)),
            pl.BlockSpec((bt, E), lambda t, e, j: (t, 0)),
            pl.BlockSpec((bt, M), lambda t, e, j: (t, 0)),
        ],
        out_specs=pl.BlockSpec((bt, M), lambda t, e, j: (t, 0)),
        out_shape=jax.ShapeDtypeStruct((T, M), jnp.float32),
        compiler_params=pltpu.CompilerParams(
            dimension_semantics=("arbitrary", "arbitrary", "arbitrary")),
    )(z, We, bWe, xv, W2, bW2, gates, x1)


def kernel(x, gamma1, beta1, Wq, bq, Wk, bk, Wv, bv, Wo, bo, gamma2, beta2,
           Wg, bg, Wn, bn, We, bWe, V, bV, W2, bW2):
    B, N, M = x.shape
    T = B * N
    NH = M // _HD
    NG = Wk.shape[1] // _HD
    E = Wg.shape[1]
    DH = We.shape[2]

    xf = x.reshape(T, M)
    g1r = gamma1.reshape(1, M)
    b1r = beta1.reshape(1, M)

    Wqkv = jnp.concatenate([Wq, Wk, Wv], axis=1)
    bqkv = jnp.concatenate([bq, bk, bv]).reshape(1, -1)

    qkv = _ln_qkv(xf, g1r, b1r, Wqkv, bqkv, bm=min(256, T),
                  bn=min(1024, Wqkv.shape[1]))
    q = qkv[:, :M].reshape(N, NH, _HD).transpose(1, 0, 2)
    k = qkv[:, M:M + NG * _HD].reshape(N, NG, _HD).transpose(1, 2, 0)
    v = qkv[:, M + NG * _HD:].reshape(N, NG, _HD).transpose(1, 0, 2)

    ao = _attention(q, k, v, bq=min(256, N))
    ao = ao.transpose(1, 0, 2).reshape(T, M)

    x1 = _proj_residual(ao, Wo, bo.reshape(1, M), xf, bm=min(256, T),
                        bn=min(1024, M))

    noise = jax.random.normal(jax.random.key(42), (T, E), dtype=jnp.float32)
    EP = 128
    pad = jnp.zeros((M, EP - E), jnp.float32)
    padb = jnp.zeros((EP - E,), jnp.float32)
    z, gates, idx8, sm8, lmoe, _hv, _ns = _gating(
        x1, gamma2.reshape(1, M), beta2.reshape(1, M),
        jnp.concatenate([Wg, pad], 1),
        jnp.concatenate([bg, padb]).reshape(1, EP),
        jnp.concatenate([Wn, pad], 1),
        jnp.concatenate([bn, padb]).reshape(1, EP), noise, bt=min(512, T))

    xv = _xv(z, V, bV.reshape(1, DH), bm=min(256, T), bn=min(1024, DH))

    out = _experts_dense(z, We, bWe.reshape(E, 1, DH), xv, W2,
                         bW2.reshape(1, M), gates, x1,
                         bt=min(512, T), bj=min(512, DH))

    return out.reshape(B, N, M), lmoe[0, 0]
